# baseline (speedup 1.0000x reference)
import functools

import jax
import jax.numpy as jnp
from jax import lax
from jax.experimental import pallas as pl
from jax.experimental.pallas import tpu as pltpu
from jax.experimental.pallas import tpu_sc as plsc

N = 10000
NPAD = 10240
E = 320000
EPAD = 327680
D_IN = 128
D_HID = 256
D_OUT = 256

CHUNK = 128
EROWS = EPAD // CHUNK
EROWS_TEC = EROWS // 16
EROWS_DEG = EROWS // 32
NROWS_TEC = NPAD // 16
IBLK = 16
CHUNK_A = 64
ACH_TEC = EPAD // CHUNK_A // 16
AIBLK = 32
DEGW = 128
HALF = 128

_vector_mesh = plsc.VectorSubcoreMesh(
    core_axis_name="core", subcore_axis_name="subcore"
)


def _sc_degree(dst2d, zeros_tab, ones_rows):
    @functools.partial(
        pl.kernel,
        out_type=jax.ShapeDtypeStruct((2, NPAD, DEGW), jnp.float32),
        mesh=_vector_mesh,
        scratch_types=[
            pltpu.VMEM_SHARED((NPAD, DEGW), jnp.float32),
            pltpu.VMEM((EROWS_DEG, CHUNK), jnp.int32),
            pltpu.VMEM((CHUNK, DEGW), jnp.float32),
        ],
    )
    def deg_kernel(dst_hbm, zeros_hbm, ones_hbm, out_hbm, table, dst_t, ones_t):
        c = lax.axis_index("core")
        s = lax.axis_index("subcore")
        pltpu.sync_copy(
            zeros_hbm.at[pl.ds(s * NROWS_TEC, NROWS_TEC)],
            table.at[pl.ds(s * NROWS_TEC, NROWS_TEC)],
        )
        pltpu.sync_copy(
            dst_hbm.at[pl.ds(c * (EROWS // 2) + s * EROWS_DEG, EROWS_DEG)],
            dst_t,
        )
        pltpu.sync_copy(ones_hbm, ones_t)
        plsc.subcore_barrier()

        @pl.loop(0, EROWS_DEG)
        def _(j):
            pltpu.sync_copy(ones_t, table.at[dst_t.at[j]], add=True)

        plsc.subcore_barrier()
        pltpu.sync_copy(
            table.at[pl.ds(s * NROWS_TEC, NROWS_TEC)],
            out_hbm.at[c, pl.ds(s * NROWS_TEC, NROWS_TEC)],
        )

    return deg_kernel(dst2d, zeros_tab, ones_rows)


def _sc_aggregate(hs, srcA, dstA):
    @functools.partial(
        pl.kernel,
        out_type=jax.ShapeDtypeStruct((2, NPAD, HALF), jnp.float32),
        mesh=_vector_mesh,
        scratch_types=[
            pltpu.VMEM_SHARED((NPAD, HALF), jnp.float32),
            pltpu.VMEM((2, AIBLK, CHUNK_A), jnp.int32),
            pltpu.VMEM((2, AIBLK, CHUNK_A), jnp.int32),
            pltpu.VMEM((4, CHUNK_A, HALF), jnp.float32),
            [pltpu.SemaphoreType.DMA] * 4,
            [pltpu.SemaphoreType.DMA] * 4,
            pltpu.SemaphoreType.DMA,
        ],
    )
    def agg_kernel(hs_hbm, src_hbm, dst_hbm, out_hbm,
                   acc, src_t, dst_t, rows, gsems, ssems, isem):
        c = lax.axis_index("core")
        s = lax.axis_index("subcore")
        hs_c = hs_hbm.at[c]

        pltpu.sync_copy(
            hs_c.at[pl.ds(s * NROWS_TEC, NROWS_TEC)],
            acc.at[pl.ds(s * NROWS_TEC, NROWS_TEC)],
        )
        plsc.subcore_barrier()

        def wait_rows(b, sem):
            pltpu.make_async_copy(hs_c.at[pl.ds(0, CHUNK_A)], rows.at[b], sem).wait()

        def src_at(m):
            return src_t.at[(m // AIBLK) % 2, m % AIBLK]

        def dst_at(m):
            return dst_t.at[(m // AIBLK) % 2, m % AIBLK]

        def prefetch_blk(blk):
            nxt = s * ACH_TEC + blk * AIBLK
            pltpu.async_copy(src_hbm.at[pl.ds(nxt, AIBLK)],
                             src_t.at[blk % 2], isem)
            pltpu.async_copy(dst_hbm.at[pl.ds(nxt, AIBLK)],
                             dst_t.at[blk % 2], isem)

        def wait_blk(blk):
            base = s * ACH_TEC
            pltpu.make_async_copy(src_hbm.at[pl.ds(base, AIBLK)],
                                  src_t.at[blk % 2], isem).wait()
            pltpu.make_async_copy(dst_hbm.at[pl.ds(base, AIBLK)],
                                  dst_t.at[blk % 2], isem).wait()

        base0 = s * ACH_TEC
        pltpu.sync_copy(src_hbm.at[pl.ds(base0, AIBLK)], src_t.at[0])
        pltpu.sync_copy(dst_hbm.at[pl.ds(base0, AIBLK)], dst_t.at[0])
        prefetch_blk(1)
        pltpu.async_copy(hs_c.at[src_at(0)], rows.at[0], gsems[0])
        pltpu.async_copy(hs_c.at[src_at(1)], rows.at[1], gsems[1])

        @pl.loop(0, ACH_TEC, step=4)
        def _(j):
            for k in range(4):
                m = j + k
                wait_rows(k, gsems[k])
                pltpu.async_copy(rows.at[k], acc.at[dst_at(m)],
                                 ssems[k], add=True)

                k2 = (k + 2) % 4

                @pl.when(m + 2 < ACH_TEC)
                def _():
                    @pl.when((m + 2) % AIBLK == 0)
                    def _():
                        wait_blk((m + 2) // AIBLK)

                        @pl.when((m + 2) // AIBLK + 1 < ACH_TEC // AIBLK)
                        def _():
                            prefetch_blk((m + 2) // AIBLK + 1)

                    @pl.when(m >= 2)
                    def _():
                        wait_rows(k2, ssems[k2])
                    pltpu.async_copy(hs_c.at[src_at(m + 2)], rows.at[k2],
                                     gsems[k2])

        for k in range(4):
            wait_rows(k, ssems[k])
        plsc.subcore_barrier()
        pltpu.sync_copy(
            acc.at[pl.ds(s * NROWS_TEC, NROWS_TEC)],
            out_hbm.at[c, pl.ds(s * NROWS_TEC, NROWS_TEC)],
        )

    return agg_kernel(hs, srcA, dstA)


BM = 1024
GRID = NPAD // BM

_HIGH = lax.Precision.HIGHEST


def _norm_col(d_ref):
    deg = d_ref[0, :, 0:1] + d_ref[1, :, 0:1] + 1.0
    return lax.rsqrt(deg)


def _tc1_body(x_ref, w_ref, d_ref, o_ref):
    norm = _norm_col(d_ref)
    p = jnp.dot(x_ref[...] * norm, w_ref[...],
                preferred_element_type=jnp.float32, precision=_HIGH)
    o_ref[0] = p[:, :HALF]
    o_ref[1] = p[:, HALF:]


def _tc_scale_matmul1(x, W1, deg):
    return pl.pallas_call(
        _tc1_body,
        grid=(GRID,),
        in_specs=[
            pl.BlockSpec((BM, D_IN), lambda i: (i, 0)),
            pl.BlockSpec((D_IN, D_HID), lambda i: (0, 0)),
            pl.BlockSpec((2, BM, DEGW), lambda i: (0, i, 0)),
        ],
        out_specs=pl.BlockSpec((2, BM, HALF), lambda i: (0, i, 0)),
        out_shape=jax.ShapeDtypeStruct((2, NPAD, HALF), jnp.float32),
    )(x, W1, deg)


def _tc2_body(a_ref, d_ref, b_ref, w_ref, o_ref):
    norm = _norm_col(d_ref)
    b = b_ref[...]
    h_lo = jax.nn.relu(a_ref[0] * norm + b[:, :HALF]) * norm
    h_hi = jax.nn.relu(a_ref[1] * norm + b[:, HALF:]) * norm
    p = (jnp.dot(h_lo, w_ref[:HALF, :],
                 preferred_element_type=jnp.float32, precision=_HIGH)
         + jnp.dot(h_hi, w_ref[HALF:, :],
                   preferred_element_type=jnp.float32, precision=_HIGH))
    o_ref[0] = p[:, :HALF]
    o_ref[1] = p[:, HALF:]


def _tc_mid(agg1, deg, b1, W2):
    return pl.pallas_call(
        _tc2_body,
        grid=(GRID,),
        in_specs=[
            pl.BlockSpec((2, BM, HALF), lambda i: (0, i, 0)),
            pl.BlockSpec((2, BM, DEGW), lambda i: (0, i, 0)),
            pl.BlockSpec((1, D_HID), lambda i: (0, 0)),
            pl.BlockSpec((D_HID, D_OUT), lambda i: (0, 0)),
        ],
        out_specs=pl.BlockSpec((2, BM, HALF), lambda i: (0, i, 0)),
        out_shape=jax.ShapeDtypeStruct((2, NPAD, HALF), jnp.float32),
    )(agg1, deg, b1, W2)


def _tc3_body(a_ref, d_ref, b_ref, o_ref):
    norm = _norm_col(d_ref)
    b = b_ref[...]
    o_ref[:, :HALF] = a_ref[0] * norm + b[:, :HALF]
    o_ref[:, HALF:] = a_ref[1] * norm + b[:, HALF:]


def _tc_final(agg2, deg, b2):
    return pl.pallas_call(
        _tc3_body,
        grid=(GRID,),
        in_specs=[
            pl.BlockSpec((2, BM, HALF), lambda i: (0, i, 0)),
            pl.BlockSpec((2, BM, DEGW), lambda i: (0, i, 0)),
            pl.BlockSpec((1, D_OUT), lambda i: (0, 0)),
        ],
        out_specs=pl.BlockSpec((BM, D_OUT), lambda i: (i, 0)),
        out_shape=jax.ShapeDtypeStruct((NPAD, D_OUT), jnp.float32),
    )(agg2, deg, b2)


def kernel(x, edge_index, W1, b1, W2, b2):
    pad = jnp.full((EPAD - E,), NPAD - 1, jnp.int32)
    src2d = jnp.concatenate([edge_index[0], pad]).reshape(EROWS, CHUNK)
    dst2d = jnp.concatenate([edge_index[1], pad]).reshape(EROWS, CHUNK)
    x_pad = jnp.zeros((NPAD, D_IN), jnp.float32).at[:N].set(x)
    zeros_tab = jnp.zeros((NPAD, DEGW), jnp.float32)
    ones_rows = jnp.ones((CHUNK, DEGW), jnp.float32)

    srcA = jnp.concatenate([edge_index[0], pad]).reshape(EPAD // CHUNK_A, CHUNK_A)
    dstA = jnp.concatenate([edge_index[1], pad]).reshape(EPAD // CHUNK_A, CHUNK_A)

    deg = _sc_degree(dst2d, zeros_tab, ones_rows)
    hs1 = _tc_scale_matmul1(x_pad, W1, deg)
    agg1 = _sc_aggregate(hs1, srcA, dstA)
    hs2 = _tc_mid(agg1, deg, b1.reshape(1, D_HID), W2)
    agg2 = _sc_aggregate(hs2, srcA, dstA)
    return _tc_final(agg2, deg, b2.reshape(1, D_OUT))[:N]


# speedup vs baseline: 1.0502x; 1.0502x over previous
import dataclasses
import functools

import jax
import jax.numpy as jnp
from jax import lax
from jax.experimental import pallas as pl
from jax.experimental.pallas import tpu as pltpu
from jax.experimental.pallas import tpu_sc as plsc

N = 10000
NPAD = 10240
E = 320000
EPAD = 327680
D_IN = 128
D_HID = 256
D_OUT = 256

CHUNK = 128
EROWS = EPAD // CHUNK
EROWS_TEC = EROWS // 16
EROWS_DEG = EROWS // 32
NROWS_TEC = NPAD // 16
DEGW = 128

HALF = 128
IBLK = 16

_vector_mesh = plsc.VectorSubcoreMesh(
    core_axis_name="core", subcore_axis_name="subcore"
)


def _sc_degree(dst2d, zeros_tab, ones_rows):
    @functools.partial(
        pl.kernel,
        out_type=jax.ShapeDtypeStruct((2, NPAD, DEGW), jnp.float32),
        mesh=_vector_mesh,
        scratch_types=[
            pltpu.VMEM_SHARED((NPAD, DEGW), jnp.float32),
            pltpu.VMEM((EROWS_DEG, CHUNK), jnp.int32),
            pltpu.VMEM((CHUNK, DEGW), jnp.float32),
        ],
    )
    def deg_kernel(dst_hbm, zeros_hbm, ones_hbm, out_hbm, table, dst_t, ones_t):
        c = lax.axis_index("core")
        s = lax.axis_index("subcore")
        pltpu.sync_copy(
            zeros_hbm.at[pl.ds(s * NROWS_TEC, NROWS_TEC)],
            table.at[pl.ds(s * NROWS_TEC, NROWS_TEC)],
        )
        pltpu.sync_copy(
            dst_hbm.at[pl.ds(c * (EROWS // 2) + s * EROWS_DEG, EROWS_DEG)],
            dst_t,
        )
        pltpu.sync_copy(ones_hbm, ones_t)
        plsc.subcore_barrier()

        @pl.loop(0, EROWS_DEG)
        def _(j):
            pltpu.sync_copy(ones_t, table.at[dst_t.at[j]], add=True)

        plsc.subcore_barrier()
        pltpu.sync_copy(
            table.at[pl.ds(s * NROWS_TEC, NROWS_TEC)],
            out_hbm.at[c, pl.ds(s * NROWS_TEC, NROWS_TEC)],
        )

    return deg_kernel(dst2d, zeros_tab, ones_rows)


def _sc_aggregate(hs, src2d, dst2d):
    @functools.partial(
        pl.kernel,
        out_type=jax.ShapeDtypeStruct((2, NPAD, HALF), jnp.float32),
        mesh=_vector_mesh,
        scratch_types=[
            pltpu.VMEM_SHARED((NPAD, HALF), jnp.float32),
            pltpu.VMEM((2, IBLK, CHUNK), jnp.int32),
            pltpu.VMEM((2, IBLK, CHUNK), jnp.int32),
            pltpu.VMEM((CHUNK, HALF), jnp.float32),
            pltpu.VMEM((CHUNK, HALF), jnp.float32),
            pltpu.SemaphoreType.DMA,
            pltpu.SemaphoreType.DMA,
            pltpu.SemaphoreType.DMA,
            pltpu.SemaphoreType.DMA,
            pltpu.SemaphoreType.DMA,
        ],
    )
    def agg_kernel(hs_hbm, src_hbm, dst_hbm, out_hbm,
                   acc, src_t, dst_t, rows0, rows1,
                   gsem0, gsem1, ssem0, ssem1, isem):
        c = lax.axis_index("core")
        s = lax.axis_index("subcore")
        hs_c = hs_hbm.at[c]
        nblk = EROWS_TEC // IBLK
        base0 = s * EROWS_TEC

        pltpu.sync_copy(
            hs_c.at[pl.ds(s * NROWS_TEC, NROWS_TEC)],
            acc.at[pl.ds(s * NROWS_TEC, NROWS_TEC)],
        )
        plsc.subcore_barrier()

        def wait_bytes(buf, sem):
            pltpu.make_async_copy(hs_c.at[pl.ds(0, CHUNK)], buf, sem).wait()

        def wait_iblk(slot):
            pltpu.make_async_copy(src_hbm.at[pl.ds(base0, IBLK)],
                                  src_t.at[slot], isem).wait()
            pltpu.make_async_copy(dst_hbm.at[pl.ds(base0, IBLK)],
                                  dst_t.at[slot], isem).wait()

        def prefetch_iblk(blk):
            nxt = base0 + blk * IBLK
            pltpu.async_copy(src_hbm.at[pl.ds(nxt, IBLK)],
                             src_t.at[blk % 2], isem)
            pltpu.async_copy(dst_hbm.at[pl.ds(nxt, IBLK)],
                             dst_t.at[blk % 2], isem)

        def src_at(j):
            return src_t.at[(j // IBLK) % 2, j % IBLK]

        def dst_at(j):
            return dst_t.at[(j // IBLK) % 2, j % IBLK]

        pltpu.sync_copy(src_hbm.at[pl.ds(base0, IBLK)], src_t.at[0])
        pltpu.sync_copy(dst_hbm.at[pl.ds(base0, IBLK)], dst_t.at[0])
        prefetch_iblk(1)
        pltpu.async_copy(hs_c.at[src_at(0)], rows0, gsem0)

        @pl.loop(0, EROWS_TEC, step=2)
        def _(j):
            @pl.when(jnp.logical_and(
                j % IBLK == 2,
                jnp.logical_and(j // IBLK >= 1, j // IBLK + 1 < nblk)))
            def _():
                prefetch_iblk(j // IBLK + 1)

            @pl.when(j > 0)
            def _():
                wait_bytes(rows1, ssem1)
            pltpu.async_copy(hs_c.at[src_at(j + 1)], rows1, gsem1)
            wait_bytes(rows0, gsem0)
            pltpu.async_copy(rows0, acc.at[dst_at(j)], ssem0, add=True)

            @pl.when(j + 2 < EROWS_TEC)
            def _():
                wait_bytes(rows0, ssem0)

                @pl.when((j + 2) % IBLK == 0)
                def _():
                    wait_iblk((j // IBLK + 1) % 2)
                pltpu.async_copy(hs_c.at[src_at(j + 2)], rows0, gsem0)
            wait_bytes(rows1, gsem1)
            pltpu.async_copy(rows1, acc.at[dst_at(j + 1)], ssem1, add=True)

        wait_bytes(rows0, ssem0)
        wait_bytes(rows1, ssem1)
        plsc.subcore_barrier()
        pltpu.sync_copy(
            acc.at[pl.ds(s * NROWS_TEC, NROWS_TEC)],
            out_hbm.at[c, pl.ds(s * NROWS_TEC, NROWS_TEC)],
        )

    return agg_kernel(hs, src2d, dst2d)


BM = 1024
GRID = NPAD // BM

_HIGH = lax.Precision.HIGHEST


def _norm_col(d_ref):
    deg = d_ref[0, :, 0:1] + d_ref[1, :, 0:1] + 1.0
    return lax.rsqrt(deg)


def _tc1_body(x_ref, w_ref, d_ref, o_ref):
    norm = _norm_col(d_ref)
    p = jnp.dot(x_ref[...] * norm, w_ref[...],
                preferred_element_type=jnp.float32, precision=_HIGH)
    o_ref[0] = p[:, :HALF]
    o_ref[1] = p[:, HALF:]


def _tc_scale_matmul1(x, W1, deg):
    return pl.pallas_call(
        _tc1_body,
        grid=(GRID,),
        in_specs=[
            pl.BlockSpec((BM, D_IN), lambda i: (i, 0)),
            pl.BlockSpec((D_IN, D_HID), lambda i: (0, 0)),
            pl.BlockSpec((2, BM, DEGW), lambda i: (0, i, 0)),
        ],
        out_specs=pl.BlockSpec((2, BM, HALF), lambda i: (0, i, 0)),
        out_shape=jax.ShapeDtypeStruct((2, NPAD, HALF), jnp.float32),
    )(x, W1, deg)


def _tc2_body(a_ref, d_ref, b_ref, w_ref, o_ref):
    norm = _norm_col(d_ref)
    b = b_ref[...]
    h_lo = jax.nn.relu(a_ref[0] * norm + b[:, :HALF]) * norm
    h_hi = jax.nn.relu(a_ref[1] * norm + b[:, HALF:]) * norm
    p = (jnp.dot(h_lo, w_ref[:HALF, :],
                 preferred_element_type=jnp.float32, precision=_HIGH)
         + jnp.dot(h_hi, w_ref[HALF:, :],
                   preferred_element_type=jnp.float32, precision=_HIGH))
    o_ref[0] = p[:, :HALF]
    o_ref[1] = p[:, HALF:]


def _tc_mid(agg1, deg, b1, W2):
    return pl.pallas_call(
        _tc2_body,
        grid=(GRID,),
        in_specs=[
            pl.BlockSpec((2, BM, HALF), lambda i: (0, i, 0)),
            pl.BlockSpec((2, BM, DEGW), lambda i: (0, i, 0)),
            pl.BlockSpec((1, D_HID), lambda i: (0, 0)),
            pl.BlockSpec((D_HID, D_OUT), lambda i: (0, 0)),
        ],
        out_specs=pl.BlockSpec((2, BM, HALF), lambda i: (0, i, 0)),
        out_shape=jax.ShapeDtypeStruct((2, NPAD, HALF), jnp.float32),
    )(agg1, deg, b1, W2)


def _tc3_body(a_ref, d_ref, b_ref, o_ref):
    norm = _norm_col(d_ref)
    b = b_ref[...]
    o_ref[:, :HALF] = a_ref[0] * norm + b[:, :HALF]
    o_ref[:, HALF:] = a_ref[1] * norm + b[:, HALF:]


def _tc_final(agg2, deg, b2):
    return pl.pallas_call(
        _tc3_body,
        grid=(GRID,),
        in_specs=[
            pl.BlockSpec((2, BM, HALF), lambda i: (0, i, 0)),
            pl.BlockSpec((2, BM, DEGW), lambda i: (0, i, 0)),
            pl.BlockSpec((1, D_OUT), lambda i: (0, 0)),
        ],
        out_specs=pl.BlockSpec((BM, D_OUT), lambda i: (i, 0)),
        out_shape=jax.ShapeDtypeStruct((NPAD, D_OUT), jnp.float32),
    )(agg2, deg, b2)


def kernel(x, edge_index, W1, b1, W2, b2):
    pad = jnp.full((EPAD - E,), NPAD - 1, jnp.int32)
    src2d = jnp.concatenate([edge_index[0], pad]).reshape(EROWS, CHUNK)
    dst2d = jnp.concatenate([edge_index[1], pad]).reshape(EROWS, CHUNK)
    x_pad = jnp.zeros((NPAD, D_IN), jnp.float32).at[:N].set(x)
    zeros_tab = jnp.zeros((NPAD, DEGW), jnp.float32)
    ones_rows = jnp.ones((CHUNK, DEGW), jnp.float32)

    deg = _sc_degree(dst2d, zeros_tab, ones_rows)
    hs1 = _tc_scale_matmul1(x_pad, W1, deg)
    agg1 = _sc_aggregate(hs1, src2d, dst2d)
    hs2 = _tc_mid(agg1, deg, b1.reshape(1, D_HID), W2)
    agg2 = _sc_aggregate(hs2, src2d, dst2d)
    return _tc_final(agg2, deg, b2.reshape(1, D_OUT))[:N]


# speedup vs baseline: 1.0588x; 1.0082x over previous
import dataclasses
import functools

import jax
import jax.numpy as jnp
from jax import lax
from jax.experimental import pallas as pl
from jax.experimental.pallas import tpu as pltpu
from jax.experimental.pallas import tpu_sc as plsc

N = 10000
NPAD = 10240
E = 320000
EPAD = 327680
D_IN = 128
D_HID = 256
D_OUT = 256

CHUNK = 128
EROWS = EPAD // CHUNK
EROWS_TEC = EROWS // 16
EROWS_DEG = EROWS // 32
NROWS_TEC = NPAD // 16
DEGW = 128

HALF = 128
IBLK = 16

_vector_mesh = plsc.VectorSubcoreMesh(
    core_axis_name="core", subcore_axis_name="subcore"
)


def _sc_degree(dst2d, zeros_tab, ones_rows):
    @functools.partial(
        pl.kernel,
        out_type=jax.ShapeDtypeStruct((2, NPAD, DEGW), jnp.float32),
        mesh=_vector_mesh,
        scratch_types=[
            pltpu.VMEM_SHARED((NPAD, DEGW), jnp.float32),
            pltpu.VMEM((EROWS_DEG, CHUNK), jnp.int32),
            pltpu.VMEM((CHUNK, DEGW), jnp.float32),
            pltpu.SemaphoreType.DMA,
        ],
    )
    def deg_kernel(dst_hbm, zeros_hbm, ones_hbm, out_hbm, table, dst_t, ones_t,
                   dsem):
        c = lax.axis_index("core")
        s = lax.axis_index("subcore")
        pltpu.sync_copy(
            zeros_hbm.at[pl.ds(s * NROWS_TEC, NROWS_TEC)],
            table.at[pl.ds(s * NROWS_TEC, NROWS_TEC)],
        )
        pltpu.sync_copy(
            dst_hbm.at[pl.ds(c * (EROWS // 2) + s * EROWS_DEG, EROWS_DEG)],
            dst_t,
        )
        pltpu.sync_copy(ones_hbm, ones_t)
        plsc.subcore_barrier()

        def drain_one():
            pltpu.make_async_copy(zeros_hbm.at[pl.ds(0, CHUNK)],
                                  ones_t, dsem).wait()

        @pl.loop(0, EROWS_DEG)
        def _(j):
            pltpu.async_copy(ones_t, table.at[dst_t.at[j]], dsem, add=True)

            @pl.when(j >= 4)
            def _():
                drain_one()

        @pl.loop(0, 4)
        def _(j):
            drain_one()

        plsc.subcore_barrier()
        pltpu.sync_copy(
            table.at[pl.ds(s * NROWS_TEC, NROWS_TEC)],
            out_hbm.at[c, pl.ds(s * NROWS_TEC, NROWS_TEC)],
        )

    return deg_kernel(dst2d, zeros_tab, ones_rows)


def _sc_aggregate(hs, src2d, dst2d):
    @functools.partial(
        pl.kernel,
        out_type=jax.ShapeDtypeStruct((2, NPAD, HALF), jnp.float32),
        mesh=_vector_mesh,
        scratch_types=[
            pltpu.VMEM_SHARED((NPAD, HALF), jnp.float32),
            pltpu.VMEM((2, IBLK, CHUNK), jnp.int32),
            pltpu.VMEM((2, IBLK, CHUNK), jnp.int32),
            pltpu.VMEM((CHUNK, HALF), jnp.float32),
            pltpu.VMEM((CHUNK, HALF), jnp.float32),
            pltpu.SemaphoreType.DMA,
            pltpu.SemaphoreType.DMA,
            pltpu.SemaphoreType.DMA,
            pltpu.SemaphoreType.DMA,
            pltpu.SemaphoreType.DMA,
        ],
    )
    def agg_kernel(hs_hbm, src_hbm, dst_hbm, out_hbm,
                   acc, src_t, dst_t, rows0, rows1,
                   gsem0, gsem1, ssem0, ssem1, isem):
        c = lax.axis_index("core")
        s = lax.axis_index("subcore")
        hs_c = hs_hbm.at[c]
        nblk = EROWS_TEC // IBLK
        base0 = s * EROWS_TEC

        pltpu.sync_copy(
            hs_c.at[pl.ds(s * NROWS_TEC, NROWS_TEC)],
            acc.at[pl.ds(s * NROWS_TEC, NROWS_TEC)],
        )
        plsc.subcore_barrier()

        def wait_bytes(buf, sem):
            pltpu.make_async_copy(hs_c.at[pl.ds(0, CHUNK)], buf, sem).wait()

        def wait_iblk(slot):
            pltpu.make_async_copy(src_hbm.at[pl.ds(base0, IBLK)],
                                  src_t.at[slot], isem).wait()
            pltpu.make_async_copy(dst_hbm.at[pl.ds(base0, IBLK)],
                                  dst_t.at[slot], isem).wait()

        def prefetch_iblk(blk):
            nxt = base0 + blk * IBLK
            pltpu.async_copy(src_hbm.at[pl.ds(nxt, IBLK)],
                             src_t.at[blk % 2], isem)
            pltpu.async_copy(dst_hbm.at[pl.ds(nxt, IBLK)],
                             dst_t.at[blk % 2], isem)

        def src_at(j):
            return src_t.at[(j // IBLK) % 2, j % IBLK]

        def dst_at(j):
            return dst_t.at[(j // IBLK) % 2, j % IBLK]

        pltpu.sync_copy(src_hbm.at[pl.ds(base0, IBLK)], src_t.at[0])
        pltpu.sync_copy(dst_hbm.at[pl.ds(base0, IBLK)], dst_t.at[0])
        prefetch_iblk(1)
        pltpu.async_copy(hs_c.at[src_at(0)], rows0, gsem0)

        @pl.loop(0, EROWS_TEC, step=2)
        def _(j):
            @pl.when(jnp.logical_and(
                j % IBLK == 2,
                jnp.logical_and(j // IBLK >= 1, j // IBLK + 1 < nblk)))
            def _():
                prefetch_iblk(j // IBLK + 1)

            @pl.when(j > 0)
            def _():
                wait_bytes(rows1, ssem1)
            pltpu.async_copy(hs_c.at[src_at(j + 1)], rows1, gsem1)
            wait_bytes(rows0, gsem0)
            pltpu.async_copy(rows0, acc.at[dst_at(j)], ssem0, add=True)

            @pl.when(j + 2 < EROWS_TEC)
            def _():
                wait_bytes(rows0, ssem0)

                @pl.when((j + 2) % IBLK == 0)
                def _():
                    wait_iblk((j // IBLK + 1) % 2)
                pltpu.async_copy(hs_c.at[src_at(j + 2)], rows0, gsem0)
            wait_bytes(rows1, gsem1)
            pltpu.async_copy(rows1, acc.at[dst_at(j + 1)], ssem1, add=True)

        wait_bytes(rows0, ssem0)
        wait_bytes(rows1, ssem1)
        plsc.subcore_barrier()
        pltpu.sync_copy(
            acc.at[pl.ds(s * NROWS_TEC, NROWS_TEC)],
            out_hbm.at[c, pl.ds(s * NROWS_TEC, NROWS_TEC)],
        )

    return agg_kernel(hs, src2d, dst2d)


BM = 1024
GRID = NPAD // BM

_HIGH = lax.Precision.HIGHEST


def _norm_col(d_ref):
    deg = d_ref[0, :, 0:1] + d_ref[1, :, 0:1] + 1.0
    return lax.rsqrt(deg)


def _tc1_body(x_ref, w_ref, d_ref, o_ref):
    norm = _norm_col(d_ref)
    p = jnp.dot(x_ref[...] * norm, w_ref[...],
                preferred_element_type=jnp.float32, precision=_HIGH)
    o_ref[0] = p[:, :HALF]
    o_ref[1] = p[:, HALF:]


def _tc_scale_matmul1(x, W1, deg):
    return pl.pallas_call(
        _tc1_body,
        grid=(GRID,),
        in_specs=[
            pl.BlockSpec((BM, D_IN), lambda i: (i, 0)),
            pl.BlockSpec((D_IN, D_HID), lambda i: (0, 0)),
            pl.BlockSpec((2, BM, DEGW), lambda i: (0, i, 0)),
        ],
        out_specs=pl.BlockSpec((2, BM, HALF), lambda i: (0, i, 0)),
        out_shape=jax.ShapeDtypeStruct((2, NPAD, HALF), jnp.float32),
    )(x, W1, deg)


def _tc2_body(a_ref, d_ref, b_ref, w_ref, o_ref):
    norm = _norm_col(d_ref)
    b = b_ref[...]
    h_lo = jax.nn.relu(a_ref[0] * norm + b[:, :HALF]) * norm
    h_hi = jax.nn.relu(a_ref[1] * norm + b[:, HALF:]) * norm
    p = (jnp.dot(h_lo, w_ref[:HALF, :],
                 preferred_element_type=jnp.float32, precision=_HIGH)
         + jnp.dot(h_hi, w_ref[HALF:, :],
                   preferred_element_type=jnp.float32, precision=_HIGH))
    o_ref[0] = p[:, :HALF]
    o_ref[1] = p[:, HALF:]


def _tc_mid(agg1, deg, b1, W2):
    return pl.pallas_call(
        _tc2_body,
        grid=(GRID,),
        in_specs=[
            pl.BlockSpec((2, BM, HALF), lambda i: (0, i, 0)),
            pl.BlockSpec((2, BM, DEGW), lambda i: (0, i, 0)),
            pl.BlockSpec((1, D_HID), lambda i: (0, 0)),
            pl.BlockSpec((D_HID, D_OUT), lambda i: (0, 0)),
        ],
        out_specs=pl.BlockSpec((2, BM, HALF), lambda i: (0, i, 0)),
        out_shape=jax.ShapeDtypeStruct((2, NPAD, HALF), jnp.float32),
    )(agg1, deg, b1, W2)


def _tc3_body(a_ref, d_ref, b_ref, o_ref):
    norm = _norm_col(d_ref)
    b = b_ref[...]
    o_ref[:, :HALF] = a_ref[0] * norm + b[:, :HALF]
    o_ref[:, HALF:] = a_ref[1] * norm + b[:, HALF:]


BM3 = 1000


def _tc_final(agg2, deg, b2):
    return pl.pallas_call(
        _tc3_body,
        grid=(N // BM3,),
        in_specs=[
            pl.BlockSpec((2, BM3, HALF), lambda i: (0, i, 0)),
            pl.BlockSpec((2, BM3, DEGW), lambda i: (0, i, 0)),
            pl.BlockSpec((1, D_OUT), lambda i: (0, 0)),
        ],
        out_specs=pl.BlockSpec((BM3, D_OUT), lambda i: (i, 0)),
        out_shape=jax.ShapeDtypeStruct((N, D_OUT), jnp.float32),
    )(agg2, deg, b2)


def kernel(x, edge_index, W1, b1, W2, b2):
    pad = jnp.full((EPAD - E,), NPAD - 1, jnp.int32)
    src2d = jnp.concatenate([edge_index[0], pad]).reshape(EROWS, CHUNK)
    dst2d = jnp.concatenate([edge_index[1], pad]).reshape(EROWS, CHUNK)
    x_pad = jnp.zeros((NPAD, D_IN), jnp.float32).at[:N].set(x)
    zeros_tab = jnp.zeros((NPAD, DEGW), jnp.float32)
    ones_rows = jnp.ones((CHUNK, DEGW), jnp.float32)

    deg = _sc_degree(dst2d, zeros_tab, ones_rows)
    hs1 = _tc_scale_matmul1(x_pad, W1, deg)
    agg1 = _sc_aggregate(hs1, src2d, dst2d)
    hs2 = _tc_mid(agg1, deg, b1.reshape(1, D_HID), W2)
    agg2 = _sc_aggregate(hs2, src2d, dst2d)
    return _tc_final(agg2, deg, b2.reshape(1, D_OUT))
